# baseline (speedup 1.0000x reference)
import functools

import jax
import jax.numpy as jnp
from jax import lax
from jax.experimental import pallas as pl
from jax.experimental.pallas import tpu as pltpu
from jax.experimental.pallas import tpu_sc as plsc

B = 16384
EMB_DIM = 128
PHYLO_DIM = 64
FUSED_IN = 320

_NC = 2
_NS = 16
_NW = _NC * _NS
_CH = 128
_NSPLIT = 4
_ROWS = B // _NSPLIT
_BLK = 2048


def _sc_gather_body(bpw, species_hbm, sid_hbm, s_out, sidx_v, srows_v, sem):
    wid = lax.axis_index("s") * _NC + lax.axis_index("c")
    base = wid * bpw
    pltpu.sync_copy(sid_hbm.at[pl.ds(base, bpw)], sidx_v)
    copies = []
    for j in range(bpw // _CH):
        copies.append(pltpu.async_copy(
            species_hbm.at[sidx_v.at[pl.ds(j * _CH, _CH)]],
            srows_v.at[pl.ds(j * _CH, _CH)], sem))
    for c in copies:
        c.wait()
    pltpu.sync_copy(srows_v, s_out.at[pl.ds(base, bpw)])


@functools.lru_cache(maxsize=None)
def _get_sc_gather(rows):
    bpw = rows // _NW
    return pl.kernel(
        functools.partial(_sc_gather_body, bpw),
        out_type=jax.ShapeDtypeStruct((rows, EMB_DIM), jnp.float32),
        mesh=plsc.VectorSubcoreMesh(core_axis_name="c", subcore_axis_name="s"),
        scratch_types=[
            pltpu.VMEM((bpw,), jnp.int32),
            pltpu.VMEM((bpw, EMB_DIM), jnp.float32),
            pltpu.SemaphoreType.DMA,
        ],
    )


def _tc_fuse_body(s_ref, t_ref, phylo_ref, k_ref, p_ref, c_ref, o_ref,
                  w_ref, b_ref, out_ref):
    w = w_ref[...]
    ws = w[:, 0:EMB_DIM]
    wp = w[:, EMB_DIM:EMB_DIM + PHYLO_DIM]
    wt = w[:, EMB_DIM + PHYLO_DIM:FUSED_IN]
    dn = (((1,), (1,)), ((), ()))
    tax = jnp.concatenate([k_ref[0:1, :], p_ref[0:1, :],
                           c_ref[0:1, :], o_ref[0:1, :]], axis=1)
    c = lax.dot_general(tax, wt, dn,
                        preferred_element_type=jnp.float32) + b_ref[...]
    p = lax.dot_general(phylo_ref[...], wp, dn,
                        preferred_element_type=jnp.float32)
    oht = (t_ref[0] == lax.broadcasted_iota(jnp.int32, (EMB_DIM, _BLK), 0)
           ).astype(jnp.float32)
    acc = lax.dot_general(s_ref[...], ws, dn,
                          preferred_element_type=jnp.float32)
    acc += lax.dot_general(oht, p, (((0,), (0,)), ((), ())),
                           preferred_element_type=jnp.float32)
    out_ref[...] = acc + c


def _tc_fuse_chained_body(prev_ref, *rest):
    del prev_ref
    _tc_fuse_body(*rest)


_COMMON_SPECS = [
    pl.BlockSpec((EMB_DIM, PHYLO_DIM), lambda i: (0, 0)),
    pl.BlockSpec((10, 32), lambda i: (0, 0)),
    pl.BlockSpec((20, 32), lambda i: (0, 0)),
    pl.BlockSpec((30, 32), lambda i: (0, 0)),
    pl.BlockSpec((50, 32), lambda i: (0, 0)),
    pl.BlockSpec((EMB_DIM, FUSED_IN), lambda i: (0, 0)),
    pl.BlockSpec((1, EMB_DIM), lambda i: (0, 0)),
]

_GRID_H = _ROWS // _BLK

_tc_fuse_first = pl.pallas_call(
    _tc_fuse_body,
    grid=(_GRID_H,),
    in_specs=[
        pl.BlockSpec((_BLK, EMB_DIM), lambda i: (i, 0)),
        pl.BlockSpec((1, 1, _BLK), lambda i: (i, 0, 0)),
    ] + _COMMON_SPECS,
    out_specs=pl.BlockSpec((_BLK, EMB_DIM), lambda i: (i, 0)),
    out_shape=jax.ShapeDtypeStruct((B, EMB_DIM), jnp.float32),
)

def _make_chained(k):
    off = k * _GRID_H
    return pl.pallas_call(
        _tc_fuse_chained_body,
        grid=(_GRID_H,),
        in_specs=[
            pl.BlockSpec(memory_space=pl.ANY),
            pl.BlockSpec((_BLK, EMB_DIM), lambda i: (i, 0)),
            pl.BlockSpec((1, 1, _BLK), lambda i, off=off: (i + off, 0, 0)),
        ] + _COMMON_SPECS,
        out_specs=pl.BlockSpec((_BLK, EMB_DIM), lambda i, off=off: (i + off, 0)),
        out_shape=jax.ShapeDtypeStruct((B, EMB_DIM), jnp.float32),
        input_output_aliases={0: 0},
    )


_tc_fuse_chained = [_make_chained(k) for k in range(1, _NSPLIT)]


def kernel(species_ids, divergence_times, species_table, phylo_table,
           kingdom_table, phylum_table, class_table, order_table, W, b):
    ids = species_ids.astype(jnp.int32)
    gather = _get_sc_gather(_ROWS)
    s_parts = [gather(species_table, ids[k * _ROWS:(k + 1) * _ROWS])
               for k in range(_NSPLIT)]
    phylo_pad = jnp.pad(phylo_table, ((0, EMB_DIM - phylo_table.shape[0]),
                                      (0, 0)))
    times = divergence_times.astype(jnp.int32).reshape(B // _BLK, 1, _BLK)
    b2 = b[None, :]
    out = _tc_fuse_first(s_parts[0], times, phylo_pad, kingdom_table,
                         phylum_table, class_table, order_table, W, b2)
    for k in range(1, _NSPLIT):
        out = _tc_fuse_chained[k - 1](out, s_parts[k], times, phylo_pad,
                                      kingdom_table, phylum_table,
                                      class_table, order_table, W, b2)
    return out


# speedup vs baseline: 1.2159x; 1.2159x over previous
import functools

import jax
import jax.numpy as jnp
from jax import lax
from jax.experimental import pallas as pl
from jax.experimental.pallas import tpu as pltpu
from jax.experimental.pallas import tpu_sc as plsc

B = 16384
EMB_DIM = 128
PHYLO_DIM = 64
FUSED_IN = 320

_NC = 2
_NS = 16
_NW = _NC * _NS
_CH = 128
_NSPLIT = 1
_ROWS = B // _NSPLIT
_BLK = 2048


def _sc_gather_body(bpw, species_hbm, sid_hbm, s_out, sidx_v, srows_v,
                    gsem, wsem):
    wid = lax.axis_index("s") * _NC + lax.axis_index("c")
    base = wid * bpw
    pltpu.sync_copy(sid_hbm.at[pl.ds(base, bpw)], sidx_v)
    gathers = []
    for j in range(bpw // _CH):
        gathers.append(pltpu.async_copy(
            species_hbm.at[sidx_v.at[pl.ds(j * _CH, _CH)]],
            srows_v.at[pl.ds(j * _CH, _CH)], gsem))
    writes = []
    for j, g in enumerate(gathers):
        g.wait()
        writes.append(pltpu.async_copy(
            srows_v.at[pl.ds(j * _CH, _CH)],
            s_out.at[pl.ds(base + j * _CH, _CH)], wsem))
    for w in writes:
        w.wait()


@functools.lru_cache(maxsize=None)
def _get_sc_gather(rows):
    bpw = rows // _NW
    return pl.kernel(
        functools.partial(_sc_gather_body, bpw),
        out_type=jax.ShapeDtypeStruct((rows, EMB_DIM), jnp.float32),
        mesh=plsc.VectorSubcoreMesh(core_axis_name="c", subcore_axis_name="s"),
        scratch_types=[
            pltpu.VMEM((bpw,), jnp.int32),
            pltpu.VMEM((bpw, EMB_DIM), jnp.float32),
            pltpu.SemaphoreType.DMA,
            pltpu.SemaphoreType.DMA,
        ],
    )


def _tc_fuse_body(s_ref, t_ref, phylo_ref, k_ref, p_ref, c_ref, o_ref,
                  w_ref, b_ref, out_ref):
    w = w_ref[...]
    ws = w[:, 0:EMB_DIM]
    wp = w[:, EMB_DIM:EMB_DIM + PHYLO_DIM]
    wt = w[:, EMB_DIM + PHYLO_DIM:FUSED_IN]
    dn = (((1,), (1,)), ((), ()))
    tax = jnp.concatenate([k_ref[0:1, :], p_ref[0:1, :],
                           c_ref[0:1, :], o_ref[0:1, :]], axis=1)
    c = lax.dot_general(tax, wt, dn,
                        preferred_element_type=jnp.float32) + b_ref[...]
    p = lax.dot_general(phylo_ref[...], wp, dn,
                        preferred_element_type=jnp.float32)
    oht = (t_ref[0] == lax.broadcasted_iota(jnp.int32, (EMB_DIM, _BLK), 0)
           ).astype(jnp.float32)
    acc = lax.dot_general(s_ref[...], ws, dn,
                          preferred_element_type=jnp.float32)
    acc += lax.dot_general(oht, p, (((0,), (0,)), ((), ())),
                           preferred_element_type=jnp.float32)
    out_ref[...] = acc + c


def _tc_fuse_chained_body(prev_ref, *rest):
    del prev_ref
    _tc_fuse_body(*rest)


_COMMON_SPECS = [
    pl.BlockSpec((EMB_DIM, PHYLO_DIM), lambda i: (0, 0)),
    pl.BlockSpec((10, 32), lambda i: (0, 0)),
    pl.BlockSpec((20, 32), lambda i: (0, 0)),
    pl.BlockSpec((30, 32), lambda i: (0, 0)),
    pl.BlockSpec((50, 32), lambda i: (0, 0)),
    pl.BlockSpec((EMB_DIM, FUSED_IN), lambda i: (0, 0)),
    pl.BlockSpec((1, EMB_DIM), lambda i: (0, 0)),
]

_GRID_H = _ROWS // _BLK

_tc_fuse_first = pl.pallas_call(
    _tc_fuse_body,
    grid=(_GRID_H,),
    in_specs=[
        pl.BlockSpec((_BLK, EMB_DIM), lambda i: (i, 0)),
        pl.BlockSpec((1, 1, _BLK), lambda i: (i, 0, 0)),
    ] + _COMMON_SPECS,
    out_specs=pl.BlockSpec((_BLK, EMB_DIM), lambda i: (i, 0)),
    out_shape=jax.ShapeDtypeStruct((B, EMB_DIM), jnp.float32),
)

def _make_chained(k):
    off = k * _GRID_H
    return pl.pallas_call(
        _tc_fuse_chained_body,
        grid=(_GRID_H,),
        in_specs=[
            pl.BlockSpec(memory_space=pl.ANY),
            pl.BlockSpec((_BLK, EMB_DIM), lambda i: (i, 0)),
            pl.BlockSpec((1, 1, _BLK), lambda i, off=off: (i + off, 0, 0)),
        ] + _COMMON_SPECS,
        out_specs=pl.BlockSpec((_BLK, EMB_DIM), lambda i, off=off: (i + off, 0)),
        out_shape=jax.ShapeDtypeStruct((B, EMB_DIM), jnp.float32),
        input_output_aliases={0: 0},
    )


_tc_fuse_chained = [_make_chained(k) for k in range(1, _NSPLIT)]


def kernel(species_ids, divergence_times, species_table, phylo_table,
           kingdom_table, phylum_table, class_table, order_table, W, b):
    ids = species_ids.astype(jnp.int32)
    gather = _get_sc_gather(_ROWS)
    s_parts = [gather(species_table, ids[k * _ROWS:(k + 1) * _ROWS])
               for k in range(_NSPLIT)]
    phylo_pad = jnp.pad(phylo_table, ((0, EMB_DIM - phylo_table.shape[0]),
                                      (0, 0)))
    times = divergence_times.astype(jnp.int32).reshape(B // _BLK, 1, _BLK)
    b2 = b[None, :]
    out = _tc_fuse_first(s_parts[0], times, phylo_pad, kingdom_table,
                         phylum_table, class_table, order_table, W, b2)
    for k in range(1, _NSPLIT):
        out = _tc_fuse_chained[k - 1](out, s_parts[k], times, phylo_pad,
                                      kingdom_table, phylum_table,
                                      class_table, order_table, W, b2)
    return out


# speedup vs baseline: 1.2266x; 1.0087x over previous
import functools

import jax
import jax.numpy as jnp
from jax import lax
from jax.experimental import pallas as pl
from jax.experimental.pallas import tpu as pltpu
from jax.experimental.pallas import tpu_sc as plsc

B = 16384
EMB_DIM = 128
PHYLO_DIM = 64
FUSED_IN = 320

_NC = 2
_NS = 16
_NW = _NC * _NS
_CH = 128
_NSPLIT = 1
_ROWS = B // _NSPLIT
_BLK = 2048


def _sc_gather_body(bpw, species_hbm, sid_hbm, s_out, sidx_v, srows_v,
                    gsem, wsem):
    del wsem
    wid = lax.axis_index("s") * _NC + lax.axis_index("c")
    base = wid * bpw
    pltpu.sync_copy(sid_hbm.at[pl.ds(base, bpw)], sidx_v)
    gathers = []
    for j in range(bpw // _CH):
        gathers.append(pltpu.async_copy(
            species_hbm.at[sidx_v.at[pl.ds(j * _CH, _CH)]],
            srows_v.at[pl.ds(j * _CH, _CH)], gsem))
    for g in gathers:
        g.wait()
    pltpu.sync_copy(srows_v, s_out.at[pl.ds(base, bpw)])


@functools.lru_cache(maxsize=None)
def _get_sc_gather(rows):
    bpw = rows // _NW
    return pl.kernel(
        functools.partial(_sc_gather_body, bpw),
        out_type=jax.ShapeDtypeStruct((rows, EMB_DIM), jnp.float32),
        mesh=plsc.VectorSubcoreMesh(core_axis_name="c", subcore_axis_name="s"),
        scratch_types=[
            pltpu.VMEM((bpw,), jnp.int32),
            pltpu.VMEM((bpw, EMB_DIM), jnp.float32),
            pltpu.SemaphoreType.DMA,
            pltpu.SemaphoreType.DMA,
        ],
    )


NUM_PHYLO = 100


def _tc_fuse_body(s_ref, t_ref, phylo_ref, k_ref, p_ref, c_ref, o_ref,
                  w_ref, b_ref, out_ref):
    w = w_ref[...]
    ws = w[:, 0:EMB_DIM]
    wp = w[:, EMB_DIM:EMB_DIM + PHYLO_DIM]
    wt = w[:, EMB_DIM + PHYLO_DIM:FUSED_IN]
    dn = (((1,), (1,)), ((), ()))
    tax = jnp.concatenate([k_ref[0:1, :], p_ref[0:1, :],
                           c_ref[0:1, :], o_ref[0:1, :]], axis=1)
    c = lax.dot_general(tax, wt, dn,
                        preferred_element_type=jnp.float32) \
        + jnp.reshape(b_ref[...], (1, EMB_DIM))
    p = lax.dot_general(phylo_ref[...], wp, dn,
                        preferred_element_type=jnp.float32)
    t_row = jnp.reshape(t_ref[...], (1, _BLK))
    oht = (t_row == lax.broadcasted_iota(jnp.int32, (NUM_PHYLO, _BLK), 0)
           ).astype(jnp.float32)
    acc = lax.dot_general(s_ref[...], ws, dn,
                          preferred_element_type=jnp.float32)
    acc += lax.dot_general(oht, p, (((0,), (0,)), ((), ())),
                           preferred_element_type=jnp.float32)
    out_ref[...] = acc + c


def _tc_fuse_chained_body(prev_ref, *rest):
    del prev_ref
    _tc_fuse_body(*rest)


_COMMON_SPECS = [
    pl.BlockSpec((NUM_PHYLO, PHYLO_DIM), lambda i: (0, 0)),
    pl.BlockSpec((10, 32), lambda i: (0, 0)),
    pl.BlockSpec((20, 32), lambda i: (0, 0)),
    pl.BlockSpec((30, 32), lambda i: (0, 0)),
    pl.BlockSpec((50, 32), lambda i: (0, 0)),
    pl.BlockSpec((EMB_DIM, FUSED_IN), lambda i: (0, 0)),
    pl.BlockSpec((EMB_DIM,), lambda i: (0,)),
]

_GRID_H = _ROWS // _BLK

_tc_fuse_first = pl.pallas_call(
    _tc_fuse_body,
    grid=(_GRID_H,),
    in_specs=[
        pl.BlockSpec((_BLK, EMB_DIM), lambda i: (i, 0)),
        pl.BlockSpec((_BLK,), lambda i: (i,)),
    ] + _COMMON_SPECS,
    out_specs=pl.BlockSpec((_BLK, EMB_DIM), lambda i: (i, 0)),
    out_shape=jax.ShapeDtypeStruct((B, EMB_DIM), jnp.float32),
)

def _make_chained(k):
    off = k * _GRID_H
    return pl.pallas_call(
        _tc_fuse_chained_body,
        grid=(_GRID_H,),
        in_specs=[
            pl.BlockSpec(memory_space=pl.ANY),
            pl.BlockSpec((_BLK, EMB_DIM), lambda i: (i, 0)),
            pl.BlockSpec((_BLK,), lambda i, off=off: (i + off,)),
        ] + _COMMON_SPECS,
        out_specs=pl.BlockSpec((_BLK, EMB_DIM), lambda i, off=off: (i + off, 0)),
        out_shape=jax.ShapeDtypeStruct((B, EMB_DIM), jnp.float32),
        input_output_aliases={0: 0},
    )


_tc_fuse_chained = [_make_chained(k) for k in range(1, _NSPLIT)]


def kernel(species_ids, divergence_times, species_table, phylo_table,
           kingdom_table, phylum_table, class_table, order_table, W, b):
    ids = species_ids.astype(jnp.int32)
    gather = _get_sc_gather(_ROWS)
    s_parts = [gather(species_table, ids[k * _ROWS:(k + 1) * _ROWS])
               for k in range(_NSPLIT)]
    times = divergence_times.astype(jnp.int32)
    out = _tc_fuse_first(s_parts[0], times, phylo_table, kingdom_table,
                         phylum_table, class_table, order_table, W, b)
    for k in range(1, _NSPLIT):
        out = _tc_fuse_chained[k - 1](out, s_parts[k], times, phylo_table,
                                      kingdom_table, phylum_table,
                                      class_table, order_table, W, b)
    return out
